# Initial kernel scaffold; baseline (speedup 1.0000x reference)
import jax
import jax.numpy as jnp
from jax.experimental import pallas as pl


def kernel(atom_type, formal_charge, num_H, aromaticity, hybridization, chiral, bond_type, stereo, conjugated, in_ring, graph_distance, W_atom_type, W_formal_charge, W_num_H, W_aromaticity, W_hybridization, W_chiral, W_bond_type, W_stereo, W_conjugated, W_in_ring, W_graph_distance):
    raise NotImplementedError("write your pallas kernel here")


# speedup vs baseline: 2.2380x; 2.2380x over previous
import functools
import jax
import jax.numpy as jnp
from jax import lax
from jax.experimental import pallas as pl
from jax.experimental.pallas import tpu as pltpu
from jax.experimental.pallas import tpu_sc as plsc

NC, NS, L = 2, 16, 16
NW = NC * NS
NA, NB = 50000, 800000
DA, DB = 128, 64
CA, CB = 80, 400
NCHUNK_A = NA // CA
NCHUNK_B = NB // CB


def _build_fused(dst, nrows, d, terms):
    def body(r, _):
        for s in range(d // L):
            sl = pl.ds(s * L, L)
            acc = terms[0][0][terms[0][1](r), sl]
            for ref, fn in terms[1:]:
                acc = acc + ref[fn(r), sl]
            dst[pl.ds(r * d + s * L, L)] = acc
        return 0
    lax.fori_loop(0, nrows, body, 0)


def _sc_body(i_at, i_fc, i_nh, i_ar, i_hy, i_ch,
             i_bt, i_st, i_cj, i_ir, i_gd,
             w_at, w_fc, w_nh, w_ar, w_hy, w_ch,
             w_bt, w_st, w_cj, w_ir, w_gd,
             atom_out, bond_out,
             v_at, v_atf, v_fc, v_nh, v_ar, v_hy, v_ch,
             v_bt, v_st, v_cj, v_ir, v_gd,
             f1a, f2a, f1b, f2b,
             a0, a1, a2, a3, a4, a5,
             b0, b1, b2, b3, b4, v_out):
    wid = lax.axis_index("s") * NC + lax.axis_index("c")
    iota16 = lax.broadcasted_iota(jnp.int32, (L,), 0)

    pltpu.sync_copy(w_at, v_at)
    pltpu.sync_copy(w_fc, v_fc)
    pltpu.sync_copy(w_nh, v_nh)
    pltpu.sync_copy(w_ar, v_ar)
    pltpu.sync_copy(w_hy, v_hy)
    pltpu.sync_copy(w_ch, v_ch)
    pltpu.sync_copy(w_bt, v_bt)
    pltpu.sync_copy(w_st, v_st)
    pltpu.sync_copy(w_cj, v_cj)
    pltpu.sync_copy(w_ir, v_ir)
    pltpu.sync_copy(w_gd, v_gd)

    def at_row(r, _):
        for s in range(DA // L):
            v_atf[pl.ds(r * DA + s * L, L)] = v_at[r, pl.ds(s * L, L)]
        return 0
    lax.fori_loop(0, 100, at_row, 0)

    _build_fused(f1a, 72, DA, [(v_fc, lambda r: r // 9),
                               (v_nh, lambda r: r % 9)])
    _build_fused(f2a, 64, DA, [(v_ar, lambda r: r // 32),
                               (v_hy, lambda r: (r // 4) % 8),
                               (v_ch, lambda r: r % 4)])
    _build_fused(f1b, 64, DB, [(v_bt, lambda r: r // 8),
                               (v_st, lambda r: r % 8)])
    _build_fused(f2b, 128, DB, [(v_cj, lambda r: r // 64),
                                (v_ir, lambda r: (r // 32) % 2),
                                (v_gd, lambda r: r % 32)])

    nk_a = (NCHUNK_A - wid + NW - 1) // NW

    def atom_chunk(k, _):
        c = wid + NW * k
        sl_in = pl.ds(c * CA, CA)
        for src_ref, dst_ref in ((i_at, a0), (i_fc, a1), (i_nh, a2),
                                 (i_ar, a3), (i_hy, a4), (i_ch, a5)):
            pltpu.sync_copy(src_ref.at[sl_in], dst_ref)

        def group(g, _):
            sl = pl.ds(g * L, L)
            at = a0[sl] * DA
            c1 = ((a1[sl] + 1) * 9 + a2[sl]) * DA
            c2 = ((a3[sl] * 8 + a4[sl]) * 4 + a5[sl]) * DA
            rb = (g * L + iota16) * DA
            for d in range(DA):
                v = (plsc.load_gather(v_atf, [at + d])
                     + plsc.load_gather(f1a, [c1 + d])
                     + plsc.load_gather(f2a, [c2 + d]))
                plsc.store_scatter(v_out, [rb + d], v)
            return 0

        lax.fori_loop(0, CA // L, group, 0)
        pltpu.sync_copy(v_out.at[pl.ds(0, CA * DA)],
                        atom_out.at[pl.ds(c * (CA * DA), CA * DA)])
        return 0

    lax.fori_loop(0, nk_a, atom_chunk, 0)

    nk_b = (NCHUNK_B - wid + NW - 1) // NW

    def bond_chunk(k, _):
        c = wid + NW * k
        sl_in = pl.ds(c * CB, CB)
        for src_ref, dst_ref in ((i_bt, b0), (i_st, b1), (i_cj, b2),
                                 (i_ir, b3), (i_gd, b4)):
            pltpu.sync_copy(src_ref.at[sl_in], dst_ref)

        def group(g, _):
            sl = pl.ds(g * L, L)
            c1 = (b0[sl] * 8 + b1[sl]) * DB
            c2 = ((b2[sl] * 2 + b3[sl]) * 32 + b4[sl]) * DB
            rb = (g * L + iota16) * DB
            for d in range(DB):
                v = (plsc.load_gather(f1b, [c1 + d])
                     + plsc.load_gather(f2b, [c2 + d]))
                plsc.store_scatter(v_out, [rb + d], v)
            return 0

        lax.fori_loop(0, CB // L, group, 0)
        pltpu.sync_copy(v_out, bond_out.at[pl.ds(c * (CB * DB), CB * DB)])
        return 0

    lax.fori_loop(0, nk_b, bond_chunk, 0)


_sc_call = functools.partial(
    pl.kernel,
    out_type=(jax.ShapeDtypeStruct((NA * DA,), jnp.float32),
              jax.ShapeDtypeStruct((NB * DB,), jnp.float32)),
    mesh=plsc.VectorSubcoreMesh(core_axis_name="c", subcore_axis_name="s",
                                num_cores=NC, num_subcores=NS),
    compiler_params=pltpu.CompilerParams(needs_layout_passes=False),
    scratch_types=[
        pltpu.VMEM((100, DA), jnp.float32),
        pltpu.VMEM((100 * DA,), jnp.float32),
        pltpu.VMEM((8, DA), jnp.float32),
        pltpu.VMEM((9, DA), jnp.float32),
        pltpu.VMEM((2, DA), jnp.float32),
        pltpu.VMEM((8, DA), jnp.float32),
        pltpu.VMEM((4, DA), jnp.float32),
        pltpu.VMEM((8, DB), jnp.float32),
        pltpu.VMEM((8, DB), jnp.float32),
        pltpu.VMEM((2, DB), jnp.float32),
        pltpu.VMEM((2, DB), jnp.float32),
        pltpu.VMEM((32, DB), jnp.float32),
        pltpu.VMEM((72 * DA,), jnp.float32),
        pltpu.VMEM((64 * DA,), jnp.float32),
        pltpu.VMEM((64 * DB,), jnp.float32),
        pltpu.VMEM((128 * DB,), jnp.float32),
        pltpu.VMEM((CA,), jnp.int32),
        pltpu.VMEM((CA,), jnp.int32),
        pltpu.VMEM((CA,), jnp.int32),
        pltpu.VMEM((CA,), jnp.int32),
        pltpu.VMEM((CA,), jnp.int32),
        pltpu.VMEM((CA,), jnp.int32),
        pltpu.VMEM((CB,), jnp.int32),
        pltpu.VMEM((CB,), jnp.int32),
        pltpu.VMEM((CB,), jnp.int32),
        pltpu.VMEM((CB,), jnp.int32),
        pltpu.VMEM((CB,), jnp.int32),
        pltpu.VMEM((CB * DB,), jnp.float32),
    ],
)(_sc_body)


@jax.jit
def kernel(atom_type, formal_charge, num_H, aromaticity, hybridization,
           chiral, bond_type, stereo, conjugated, in_ring, graph_distance,
           W_atom_type, W_formal_charge, W_num_H, W_aromaticity,
           W_hybridization, W_chiral, W_bond_type, W_stereo, W_conjugated,
           W_in_ring, W_graph_distance):
    ii = [atom_type, formal_charge, num_H, aromaticity, hybridization,
          chiral, bond_type, stereo, conjugated, in_ring, graph_distance]
    ii = [x.astype(jnp.int32) for x in ii]
    atom_flat, bond_flat = _sc_call(
        *ii,
        W_atom_type, W_formal_charge, W_num_H, W_aromaticity,
        W_hybridization, W_chiral,
        W_bond_type, W_stereo, W_conjugated, W_in_ring, W_graph_distance)
    return atom_flat.reshape(NA, DA), bond_flat.reshape(NB, DB)


# speedup vs baseline: 2.7816x; 1.2429x over previous
import functools
import jax
import jax.numpy as jnp
from jax import lax
from jax.experimental import pallas as pl
from jax.experimental.pallas import tpu as pltpu
from jax.experimental.pallas import tpu_sc as plsc

NC, NS, L = 2, 16, 16
NW = NC * NS
NA, NB = 50000, 800000
DA, DB = 128, 64
CA, CB = 80, 400
NCHUNK_A = NA // CA
NCHUNK_B = NB // CB


def _build_fused(dst, nrows, d, terms):
    def body(r, _):
        for s in range(d // L):
            sl = pl.ds(s * L, L)
            acc = terms[0][0][terms[0][1](r), sl]
            for ref, fn in terms[1:]:
                acc = acc + ref[fn(r), sl]
            dst[pl.ds(r * d + s * L, L)] = acc
        return 0
    lax.fori_loop(0, nrows, body, 0)


def _sc_body(i_at, i_fc, i_nh, i_ar, i_hy, i_ch,
             i_bt, i_st, i_cj, i_ir, i_gd,
             w_at, w_fc, w_nh, w_ar, w_hy, w_ch,
             w_bt, w_st, w_cj, w_ir, w_gd,
             atom_out, bond_out,
             v_at, v_atf, v_fc, v_nh, v_ar, v_hy, v_ch,
             v_bt, v_st, v_cj, v_ir, v_gd,
             f1a, f2a, f1b, f2b,
             a0, a1, a2, a3, a4, a5,
             b0, b1, b2, b3, b4, v_out):
    wid = lax.axis_index("s") * NC + lax.axis_index("c")
    iota16 = lax.broadcasted_iota(jnp.int32, (L,), 0)

    pltpu.sync_copy(w_at, v_at)
    pltpu.sync_copy(w_fc, v_fc)
    pltpu.sync_copy(w_nh, v_nh)
    pltpu.sync_copy(w_ar, v_ar)
    pltpu.sync_copy(w_hy, v_hy)
    pltpu.sync_copy(w_ch, v_ch)
    pltpu.sync_copy(w_bt, v_bt)
    pltpu.sync_copy(w_st, v_st)
    pltpu.sync_copy(w_cj, v_cj)
    pltpu.sync_copy(w_ir, v_ir)
    pltpu.sync_copy(w_gd, v_gd)

    def at_row(r, _):
        for s in range(DA // L):
            v_atf[pl.ds(r * DA + s * L, L)] = v_at[r, pl.ds(s * L, L)]
        return 0
    lax.fori_loop(0, 100, at_row, 0)

    _build_fused(f1a, 72, DA, [(v_fc, lambda r: r // 9),
                               (v_nh, lambda r: r % 9)])
    _build_fused(f2a, 64, DA, [(v_ar, lambda r: r // 32),
                               (v_hy, lambda r: (r // 4) % 8),
                               (v_ch, lambda r: r % 4)])
    _build_fused(f1b, 64, DB, [(v_bt, lambda r: r // 8),
                               (v_st, lambda r: r % 8)])
    _build_fused(f2b, 128, DB, [(v_cj, lambda r: r // 64),
                                (v_ir, lambda r: (r // 32) % 2),
                                (v_gd, lambda r: r % 32)])

    nk_a = (NCHUNK_A - wid + NW - 1) // NW

    def atom_chunk(k, _):
        c = wid + NW * k
        sl_in = pl.ds(c * CA, CA)
        for src_ref, dst_ref in ((i_at, a0), (i_fc, a1), (i_nh, a2),
                                 (i_ar, a3), (i_hy, a4), (i_ch, a5)):
            pltpu.sync_copy(src_ref.at[sl_in], dst_ref)

        @plsc.parallel_loop(0, CA // L, unroll=2)
        def group(g):
            sl = pl.ds(g * L, L)
            at = a0[sl] * DA
            c1 = ((a1[sl] + 1) * 9 + a2[sl]) * DA
            c2 = ((a3[sl] * 8 + a4[sl]) * 4 + a5[sl]) * DA
            rb = (g * L + iota16) * DA
            for d in range(DA):
                v = (plsc.load_gather(v_atf, [at + d])
                     + plsc.load_gather(f1a, [c1 + d])
                     + plsc.load_gather(f2a, [c2 + d]))
                plsc.store_scatter(v_out, [rb + d], v)

        pltpu.sync_copy(v_out.at[pl.ds(0, CA * DA)],
                        atom_out.at[pl.ds(c * (CA * DA), CA * DA)])
        return 0

    lax.fori_loop(0, nk_a, atom_chunk, 0)

    nk_b = (NCHUNK_B - wid + NW - 1) // NW

    def bond_chunk(k, _):
        c = wid + NW * k
        sl_in = pl.ds(c * CB, CB)
        for src_ref, dst_ref in ((i_bt, b0), (i_st, b1), (i_cj, b2),
                                 (i_ir, b3), (i_gd, b4)):
            pltpu.sync_copy(src_ref.at[sl_in], dst_ref)

        @plsc.parallel_loop(0, CB // L, unroll=2)
        def group(g):
            sl = pl.ds(g * L, L)
            c1 = (b0[sl] * 8 + b1[sl]) * DB
            c2 = ((b2[sl] * 2 + b3[sl]) * 32 + b4[sl]) * DB
            rb = (g * L + iota16) * DB
            for d in range(DB):
                v = (plsc.load_gather(f1b, [c1 + d])
                     + plsc.load_gather(f2b, [c2 + d]))
                plsc.store_scatter(v_out, [rb + d], v)

        pltpu.sync_copy(v_out, bond_out.at[pl.ds(c * (CB * DB), CB * DB)])
        return 0

    lax.fori_loop(0, nk_b, bond_chunk, 0)


_sc_call = functools.partial(
    pl.kernel,
    out_type=(jax.ShapeDtypeStruct((NA * DA,), jnp.float32),
              jax.ShapeDtypeStruct((NB * DB,), jnp.float32)),
    mesh=plsc.VectorSubcoreMesh(core_axis_name="c", subcore_axis_name="s",
                                num_cores=NC, num_subcores=NS),
    compiler_params=pltpu.CompilerParams(needs_layout_passes=False),
    scratch_types=[
        pltpu.VMEM((100, DA), jnp.float32),
        pltpu.VMEM((100 * DA,), jnp.float32),
        pltpu.VMEM((8, DA), jnp.float32),
        pltpu.VMEM((9, DA), jnp.float32),
        pltpu.VMEM((2, DA), jnp.float32),
        pltpu.VMEM((8, DA), jnp.float32),
        pltpu.VMEM((4, DA), jnp.float32),
        pltpu.VMEM((8, DB), jnp.float32),
        pltpu.VMEM((8, DB), jnp.float32),
        pltpu.VMEM((2, DB), jnp.float32),
        pltpu.VMEM((2, DB), jnp.float32),
        pltpu.VMEM((32, DB), jnp.float32),
        pltpu.VMEM((72 * DA,), jnp.float32),
        pltpu.VMEM((64 * DA,), jnp.float32),
        pltpu.VMEM((64 * DB,), jnp.float32),
        pltpu.VMEM((128 * DB,), jnp.float32),
        pltpu.VMEM((CA,), jnp.int32),
        pltpu.VMEM((CA,), jnp.int32),
        pltpu.VMEM((CA,), jnp.int32),
        pltpu.VMEM((CA,), jnp.int32),
        pltpu.VMEM((CA,), jnp.int32),
        pltpu.VMEM((CA,), jnp.int32),
        pltpu.VMEM((CB,), jnp.int32),
        pltpu.VMEM((CB,), jnp.int32),
        pltpu.VMEM((CB,), jnp.int32),
        pltpu.VMEM((CB,), jnp.int32),
        pltpu.VMEM((CB,), jnp.int32),
        pltpu.VMEM((CB * DB,), jnp.float32),
    ],
)(_sc_body)


@jax.jit
def kernel(atom_type, formal_charge, num_H, aromaticity, hybridization,
           chiral, bond_type, stereo, conjugated, in_ring, graph_distance,
           W_atom_type, W_formal_charge, W_num_H, W_aromaticity,
           W_hybridization, W_chiral, W_bond_type, W_stereo, W_conjugated,
           W_in_ring, W_graph_distance):
    ii = [atom_type, formal_charge, num_H, aromaticity, hybridization,
          chiral, bond_type, stereo, conjugated, in_ring, graph_distance]
    ii = [x.astype(jnp.int32) for x in ii]
    atom_flat, bond_flat = _sc_call(
        *ii,
        W_atom_type, W_formal_charge, W_num_H, W_aromaticity,
        W_hybridization, W_chiral,
        W_bond_type, W_stereo, W_conjugated, W_in_ring, W_graph_distance)
    return atom_flat.reshape(NA, DA), bond_flat.reshape(NB, DB)


# speedup vs baseline: 5.5123x; 1.9817x over previous
import functools
import jax
import jax.numpy as jnp
from jax import lax
from jax.experimental import pallas as pl
from jax.experimental.pallas import tpu as pltpu
from jax.experimental.pallas import tpu_sc as plsc

NC, NS, L = 2, 16, 16
NW = NC * NS
NA, NB = 50000, 800000
DA, DB = 128, 64
CA, CB = 80, 400
SA, SB = DA + 1, DB + 1
NCHUNK_A = NA // CA
NCHUNK_B = NB // CB


def _build_fused(dst, nrows, d, stride, terms):
    def body(r, _):
        for s in range(d // L):
            sl = pl.ds(s * L, L)
            acc = terms[0][0][terms[0][1](r), sl]
            for ref, fn in terms[1:]:
                acc = acc + ref[fn(r), sl]
            dst[pl.ds(r * stride + s * L, L)] = acc
        return 0
    lax.fori_loop(0, nrows, body, 0)


def _sc_body(i_at, i_fc, i_nh, i_ar, i_hy, i_ch,
             i_bt, i_st, i_cj, i_ir, i_gd,
             w_at, w_fc, w_nh, w_ar, w_hy, w_ch,
             w_bt, w_st, w_cj, w_ir, w_gd,
             atom_out, bond_out,
             v_at, v_atf, v_fc, v_nh, v_ar, v_hy, v_ch,
             v_bt, v_st, v_cj, v_ir, v_gd,
             f1a, f2a, f1b, f2b,
             a0, a1, a2, a3, a4, a5,
             b0, b1, b2, b3, b4, v_out):
    wid = lax.axis_index("s") * NC + lax.axis_index("c")
    iota16 = lax.broadcasted_iota(jnp.int32, (L,), 0)

    pltpu.sync_copy(w_at, v_at)
    pltpu.sync_copy(w_fc, v_fc)
    pltpu.sync_copy(w_nh, v_nh)
    pltpu.sync_copy(w_ar, v_ar)
    pltpu.sync_copy(w_hy, v_hy)
    pltpu.sync_copy(w_ch, v_ch)
    pltpu.sync_copy(w_bt, v_bt)
    pltpu.sync_copy(w_st, v_st)
    pltpu.sync_copy(w_cj, v_cj)
    pltpu.sync_copy(w_ir, v_ir)
    pltpu.sync_copy(w_gd, v_gd)

    def at_row(r, _):
        for s in range(DA // L):
            v_atf[pl.ds(r * SA + s * L, L)] = v_at[r, pl.ds(s * L, L)]
        return 0
    lax.fori_loop(0, 100, at_row, 0)

    _build_fused(f1a, 72, DA, SA, [(v_fc, lambda r: r // 9),
                               (v_nh, lambda r: r % 9)])
    _build_fused(f2a, 64, DA, SA, [(v_ar, lambda r: r // 32),
                               (v_hy, lambda r: (r // 4) % 8),
                               (v_ch, lambda r: r % 4)])
    _build_fused(f1b, 64, DB, SB, [(v_bt, lambda r: r // 8),
                               (v_st, lambda r: r % 8)])
    _build_fused(f2b, 128, DB, SB, [(v_cj, lambda r: r // 64),
                                (v_ir, lambda r: (r // 32) % 2),
                                (v_gd, lambda r: r % 32)])

    nk_a = (NCHUNK_A - wid + NW - 1) // NW

    def atom_chunk(k, _):
        c = wid + NW * k
        sl_in = pl.ds(c * CA, CA)
        for src_ref, dst_ref in ((i_at, a0), (i_fc, a1), (i_nh, a2),
                                 (i_ar, a3), (i_hy, a4), (i_ch, a5)):
            pltpu.sync_copy(src_ref.at[sl_in], dst_ref)

        @plsc.parallel_loop(0, CA // L)
        def group(g):
            sl = pl.ds(g * L, L)
            at = a0[sl] * SA
            c1 = ((a1[sl] + 1) * 9 + a2[sl]) * SA
            c2 = ((a3[sl] * 8 + a4[sl]) * 4 + a5[sl]) * SA
            rb = (g * L + iota16) * DA

            @plsc.parallel_loop(0, DA, unroll=8)
            def dloop(d):
                v = (plsc.load_gather(v_atf, [at + d])
                     + plsc.load_gather(f1a, [c1 + d])
                     + plsc.load_gather(f2a, [c2 + d]))
                plsc.store_scatter(v_out, [rb + d], v)

        pltpu.sync_copy(v_out.at[pl.ds(0, CA * DA)],
                        atom_out.at[pl.ds(c * (CA * DA), CA * DA)])
        return 0

    lax.fori_loop(0, nk_a, atom_chunk, 0)

    nk_b = (NCHUNK_B - wid + NW - 1) // NW

    def bond_chunk(k, _):
        c = wid + NW * k
        sl_in = pl.ds(c * CB, CB)
        for src_ref, dst_ref in ((i_bt, b0), (i_st, b1), (i_cj, b2),
                                 (i_ir, b3), (i_gd, b4)):
            pltpu.sync_copy(src_ref.at[sl_in], dst_ref)

        @plsc.parallel_loop(0, CB // L)
        def group(g):
            sl = pl.ds(g * L, L)
            c1 = (b0[sl] * 8 + b1[sl]) * SB
            c2 = ((b2[sl] * 2 + b3[sl]) * 32 + b4[sl]) * SB
            rb = (g * L + iota16) * DB

            @plsc.parallel_loop(0, DB, unroll=8)
            def dloop(d):
                v = (plsc.load_gather(f1b, [c1 + d])
                     + plsc.load_gather(f2b, [c2 + d]))
                plsc.store_scatter(v_out, [rb + d], v)

        pltpu.sync_copy(v_out, bond_out.at[pl.ds(c * (CB * DB), CB * DB)])
        return 0

    lax.fori_loop(0, nk_b, bond_chunk, 0)


_sc_call = functools.partial(
    pl.kernel,
    out_type=(jax.ShapeDtypeStruct((NA * DA,), jnp.float32),
              jax.ShapeDtypeStruct((NB * DB,), jnp.float32)),
    mesh=plsc.VectorSubcoreMesh(core_axis_name="c", subcore_axis_name="s",
                                num_cores=NC, num_subcores=NS),
    compiler_params=pltpu.CompilerParams(needs_layout_passes=False),
    scratch_types=[
        pltpu.VMEM((100, DA), jnp.float32),
        pltpu.VMEM((100 * SA,), jnp.float32),
        pltpu.VMEM((8, DA), jnp.float32),
        pltpu.VMEM((9, DA), jnp.float32),
        pltpu.VMEM((2, DA), jnp.float32),
        pltpu.VMEM((8, DA), jnp.float32),
        pltpu.VMEM((4, DA), jnp.float32),
        pltpu.VMEM((8, DB), jnp.float32),
        pltpu.VMEM((8, DB), jnp.float32),
        pltpu.VMEM((2, DB), jnp.float32),
        pltpu.VMEM((2, DB), jnp.float32),
        pltpu.VMEM((32, DB), jnp.float32),
        pltpu.VMEM((72 * SA,), jnp.float32),
        pltpu.VMEM((64 * SA,), jnp.float32),
        pltpu.VMEM((64 * SB,), jnp.float32),
        pltpu.VMEM((128 * SB,), jnp.float32),
        pltpu.VMEM((CA,), jnp.int32),
        pltpu.VMEM((CA,), jnp.int32),
        pltpu.VMEM((CA,), jnp.int32),
        pltpu.VMEM((CA,), jnp.int32),
        pltpu.VMEM((CA,), jnp.int32),
        pltpu.VMEM((CA,), jnp.int32),
        pltpu.VMEM((CB,), jnp.int32),
        pltpu.VMEM((CB,), jnp.int32),
        pltpu.VMEM((CB,), jnp.int32),
        pltpu.VMEM((CB,), jnp.int32),
        pltpu.VMEM((CB,), jnp.int32),
        pltpu.VMEM((CB * DB,), jnp.float32),
    ],
)(_sc_body)


@jax.jit
def kernel(atom_type, formal_charge, num_H, aromaticity, hybridization,
           chiral, bond_type, stereo, conjugated, in_ring, graph_distance,
           W_atom_type, W_formal_charge, W_num_H, W_aromaticity,
           W_hybridization, W_chiral, W_bond_type, W_stereo, W_conjugated,
           W_in_ring, W_graph_distance):
    ii = [atom_type, formal_charge, num_H, aromaticity, hybridization,
          chiral, bond_type, stereo, conjugated, in_ring, graph_distance]
    ii = [x.astype(jnp.int32) for x in ii]
    atom_flat, bond_flat = _sc_call(
        *ii,
        W_atom_type, W_formal_charge, W_num_H, W_aromaticity,
        W_hybridization, W_chiral,
        W_bond_type, W_stereo, W_conjugated, W_in_ring, W_graph_distance)
    return atom_flat.reshape(NA, DA), bond_flat.reshape(NB, DB)


# speedup vs baseline: 9.0578x; 1.6432x over previous
import functools
import jax
import jax.numpy as jnp
from jax import lax
from jax.experimental import pallas as pl
from jax.experimental.pallas import tpu as pltpu
from jax.experimental.pallas import tpu_sc as plsc

NC, NS, L = 2, 16, 16
NW = NC * NS
NA, NB = 50000, 800000
DA, DB = 128, 64
CA, CB = 80, 400
SA, SB = DA, DB
NCHUNK_A = NA // CA
NCHUNK_B = NB // CB


def _build_fused(dst, nrows, d, stride, terms):
    def body(r, _):
        for s in range(d // L):
            sl = pl.ds(s * L, L)
            acc = terms[0][0][terms[0][1](r), sl]
            for ref, fn in terms[1:]:
                acc = acc + ref[fn(r), sl]
            dst[pl.ds(r * stride + s * L, L)] = acc
        return 0
    lax.fori_loop(0, nrows, body, 0)


def _sc_body(i_at, i_fc, i_nh, i_ar, i_hy, i_ch,
             i_bt, i_st, i_cj, i_ir, i_gd,
             w_at, w_fc, w_nh, w_ar, w_hy, w_ch,
             w_bt, w_st, w_cj, w_ir, w_gd,
             atom_out, bond_out,
             v_at, v_atf, v_fc, v_nh, v_ar, v_hy, v_ch,
             v_bt, v_st, v_cj, v_ir, v_gd,
             f1a, f2a, f1b, f2b,
             a0, a1, a2, a3, a4, a5,
             b0, b1, b2, b3, b4, v_out):
    wid = lax.axis_index("s") * NC + lax.axis_index("c")
    iota16 = lax.broadcasted_iota(jnp.int32, (L,), 0)

    pltpu.sync_copy(w_at, v_at)
    pltpu.sync_copy(w_fc, v_fc)
    pltpu.sync_copy(w_nh, v_nh)
    pltpu.sync_copy(w_ar, v_ar)
    pltpu.sync_copy(w_hy, v_hy)
    pltpu.sync_copy(w_ch, v_ch)
    pltpu.sync_copy(w_bt, v_bt)
    pltpu.sync_copy(w_st, v_st)
    pltpu.sync_copy(w_cj, v_cj)
    pltpu.sync_copy(w_ir, v_ir)
    pltpu.sync_copy(w_gd, v_gd)

    def at_row(r, _):
        for s in range(DA // L):
            v_atf[pl.ds(r * SA + s * L, L)] = v_at[r, pl.ds(s * L, L)]
        return 0
    lax.fori_loop(0, 100, at_row, 0)

    _build_fused(f1a, 72, DA, SA, [(v_fc, lambda r: r // 9),
                               (v_nh, lambda r: r % 9)])
    _build_fused(f2a, 64, DA, SA, [(v_ar, lambda r: r // 32),
                               (v_hy, lambda r: (r // 4) % 8),
                               (v_ch, lambda r: r % 4)])
    _build_fused(f1b, 64, DB, SB, [(v_bt, lambda r: r // 8),
                               (v_st, lambda r: r % 8)])
    _build_fused(f2b, 128, DB, SB, [(v_cj, lambda r: r // 64),
                                (v_ir, lambda r: (r // 32) % 2),
                                (v_gd, lambda r: r % 32)])

    nk_a = (NCHUNK_A - wid + NW - 1) // NW

    def atom_chunk(k, _):
        c = wid + NW * k
        sl_in = pl.ds(c * CA, CA)
        for src_ref, dst_ref in ((i_at, a0), (i_fc, a1), (i_nh, a2),
                                 (i_ar, a3), (i_hy, a4), (i_ch, a5)):
            pltpu.sync_copy(src_ref.at[sl_in], dst_ref)

        @plsc.parallel_loop(0, CA // L)
        def group(g):
            sl = pl.ds(g * L, L)
            atv = a0[sl] * SA
            c1v = ((a1[sl] + 1) * 9 + a2[sl]) * SA
            c2v = ((a3[sl] * 8 + a4[sl]) * 4 + a5[sl]) * SA
            for j in range(L):
                s0 = jnp.full((L,), atv[j], jnp.int32)
                s1 = jnp.full((L,), c1v[j], jnp.int32)
                s2 = jnp.full((L,), c2v[j], jnp.int32)
                base = (g * L + j) * DA
                for k in range(DA // L):
                    ik = iota16 + (k * L)
                    v = (plsc.load_gather(v_atf, [s0 + ik])
                         + plsc.load_gather(f1a, [s1 + ik])
                         + plsc.load_gather(f2a, [s2 + ik]))
                    v_out[pl.ds(base + k * L, L)] = v

        pltpu.sync_copy(v_out.at[pl.ds(0, CA * DA)],
                        atom_out.at[pl.ds(c * (CA * DA), CA * DA)])
        return 0

    lax.fori_loop(0, nk_a, atom_chunk, 0)

    nk_b = (NCHUNK_B - wid + NW - 1) // NW

    def bond_chunk(k, _):
        c = wid + NW * k
        sl_in = pl.ds(c * CB, CB)
        for src_ref, dst_ref in ((i_bt, b0), (i_st, b1), (i_cj, b2),
                                 (i_ir, b3), (i_gd, b4)):
            pltpu.sync_copy(src_ref.at[sl_in], dst_ref)

        @plsc.parallel_loop(0, CB // L)
        def group(g):
            sl = pl.ds(g * L, L)
            c1v = (b0[sl] * 8 + b1[sl]) * SB
            c2v = ((b2[sl] * 2 + b3[sl]) * 32 + b4[sl]) * SB
            for j in range(L):
                s1 = jnp.full((L,), c1v[j], jnp.int32)
                s2 = jnp.full((L,), c2v[j], jnp.int32)
                base = (g * L + j) * DB
                for k in range(DB // L):
                    ik = iota16 + (k * L)
                    v = (plsc.load_gather(f1b, [s1 + ik])
                         + plsc.load_gather(f2b, [s2 + ik]))
                    v_out[pl.ds(base + k * L, L)] = v

        pltpu.sync_copy(v_out, bond_out.at[pl.ds(c * (CB * DB), CB * DB)])
        return 0

    lax.fori_loop(0, nk_b, bond_chunk, 0)


_sc_call = functools.partial(
    pl.kernel,
    out_type=(jax.ShapeDtypeStruct((NA * DA,), jnp.float32),
              jax.ShapeDtypeStruct((NB * DB,), jnp.float32)),
    mesh=plsc.VectorSubcoreMesh(core_axis_name="c", subcore_axis_name="s",
                                num_cores=NC, num_subcores=NS),
    compiler_params=pltpu.CompilerParams(needs_layout_passes=False),
    scratch_types=[
        pltpu.VMEM((100, DA), jnp.float32),
        pltpu.VMEM((100 * SA,), jnp.float32),
        pltpu.VMEM((8, DA), jnp.float32),
        pltpu.VMEM((9, DA), jnp.float32),
        pltpu.VMEM((2, DA), jnp.float32),
        pltpu.VMEM((8, DA), jnp.float32),
        pltpu.VMEM((4, DA), jnp.float32),
        pltpu.VMEM((8, DB), jnp.float32),
        pltpu.VMEM((8, DB), jnp.float32),
        pltpu.VMEM((2, DB), jnp.float32),
        pltpu.VMEM((2, DB), jnp.float32),
        pltpu.VMEM((32, DB), jnp.float32),
        pltpu.VMEM((72 * SA,), jnp.float32),
        pltpu.VMEM((64 * SA,), jnp.float32),
        pltpu.VMEM((64 * SB,), jnp.float32),
        pltpu.VMEM((128 * SB,), jnp.float32),
        pltpu.VMEM((CA,), jnp.int32),
        pltpu.VMEM((CA,), jnp.int32),
        pltpu.VMEM((CA,), jnp.int32),
        pltpu.VMEM((CA,), jnp.int32),
        pltpu.VMEM((CA,), jnp.int32),
        pltpu.VMEM((CA,), jnp.int32),
        pltpu.VMEM((CB,), jnp.int32),
        pltpu.VMEM((CB,), jnp.int32),
        pltpu.VMEM((CB,), jnp.int32),
        pltpu.VMEM((CB,), jnp.int32),
        pltpu.VMEM((CB,), jnp.int32),
        pltpu.VMEM((CB * DB,), jnp.float32),
    ],
)(_sc_body)


@jax.jit
def kernel(atom_type, formal_charge, num_H, aromaticity, hybridization,
           chiral, bond_type, stereo, conjugated, in_ring, graph_distance,
           W_atom_type, W_formal_charge, W_num_H, W_aromaticity,
           W_hybridization, W_chiral, W_bond_type, W_stereo, W_conjugated,
           W_in_ring, W_graph_distance):
    ii = [atom_type, formal_charge, num_H, aromaticity, hybridization,
          chiral, bond_type, stereo, conjugated, in_ring, graph_distance]
    ii = [x.astype(jnp.int32) for x in ii]
    atom_flat, bond_flat = _sc_call(
        *ii,
        W_atom_type, W_formal_charge, W_num_H, W_aromaticity,
        W_hybridization, W_chiral,
        W_bond_type, W_stereo, W_conjugated, W_in_ring, W_graph_distance)
    return atom_flat.reshape(NA, DA), bond_flat.reshape(NB, DB)


# speedup vs baseline: 10.7757x; 1.1897x over previous
import functools
import jax
import jax.numpy as jnp
from jax import lax
from jax.experimental import pallas as pl
from jax.experimental.pallas import tpu as pltpu
from jax.experimental.pallas import tpu_sc as plsc

NC, NS, L = 2, 16, 16
NW = NC * NS
NA, NB = 50000, 800000
DA, DB = 128, 64
CA, CB = 80, 256
NCHUNK_A = NA // CA
NCHUNK_B = NB // CB


def _build_fused(dst, nrows, d, terms):
    def body(r, _):
        for s in range(d // L):
            sl = pl.ds(s * L, L)
            acc = terms[0][0][terms[0][1](r), sl]
            for ref, fn in terms[1:]:
                acc = acc + ref[fn(r), sl]
            dst[pl.ds(r * d + s * L, L)] = acc
        return 0
    lax.fori_loop(0, nrows, body, 0)


def _sc_body(i_at, i_fc, i_nh, i_ar, i_hy, i_ch,
             i_bt, i_st, i_cj, i_ir, i_gd,
             w_at, w_fc, w_nh, w_ar, w_hy, w_ch,
             w_bt, w_st, w_cj, w_ir, w_gd,
             atom_out, bond_out,
             v_at, v_atf, v_fc, v_nh, v_ar, v_hy, v_ch,
             v_bt, v_st, v_cj, v_ir, v_gd,
             f1a, f2a, f1b, f2b,
             aA0, aA1, aA2, aA3, aA4, aA5,
             aB0, aB1, aB2, aB3, aB4, aB5,
             bA0, bA1, bA2, bA3, bA4,
             bB0, bB1, bB2, bB3, bB4,
             oaA, oaB, obA, obB,
             semiA, semiB, semoA, semoB):
    wid = lax.axis_index("s") * NC + lax.axis_index("c")
    iota16 = lax.broadcasted_iota(jnp.int32, (L,), 0)

    pltpu.sync_copy(w_at, v_at)
    pltpu.sync_copy(w_fc, v_fc)
    pltpu.sync_copy(w_nh, v_nh)
    pltpu.sync_copy(w_ar, v_ar)
    pltpu.sync_copy(w_hy, v_hy)
    pltpu.sync_copy(w_ch, v_ch)
    pltpu.sync_copy(w_bt, v_bt)
    pltpu.sync_copy(w_st, v_st)
    pltpu.sync_copy(w_cj, v_cj)
    pltpu.sync_copy(w_ir, v_ir)
    pltpu.sync_copy(w_gd, v_gd)

    def at_row(r, _):
        for s in range(DA // L):
            v_atf[pl.ds(r * DA + s * L, L)] = v_at[r, pl.ds(s * L, L)]
        return 0
    lax.fori_loop(0, 100, at_row, 0)

    _build_fused(f1a, 72, DA, [(v_fc, lambda r: r // 9),
                               (v_nh, lambda r: r % 9)])
    _build_fused(f2a, 64, DA, [(v_ar, lambda r: r // 32),
                               (v_hy, lambda r: (r // 4) % 8),
                               (v_ch, lambda r: r % 4)])
    _build_fused(f1b, 64, DB, [(v_bt, lambda r: r // 8),
                               (v_st, lambda r: r % 8)])
    _build_fused(f2b, 128, DB, [(v_cj, lambda r: r // 64),
                                (v_ir, lambda r: (r // 32) % 2),
                                (v_gd, lambda r: r % 32)])

    def run_phase(nchunks, C, CD, idx_srcs, bufs0, bufs1, out_ref,
                  ob0, ob1, semi0, semi1, semo0, semo1, compute):
        nk = (nchunks - wid + NW - 1) // NW

        def fire_idx(k, bufs, sem):
            sl = pl.ds((wid + NW * k) * C, C)
            for s, d in zip(idx_srcs, bufs):
                pltpu.async_copy(s.at[sl], d, sem)

        def wait_idx(bufs, sem):
            for s, d in zip(idx_srcs, bufs):
                pltpu.make_async_copy(s.at[pl.ds(0, C)], d, sem).wait()

        def fire_out(k, ob, sem):
            pltpu.async_copy(ob, out_ref.at[pl.ds((wid + NW * k) * CD, CD)],
                             sem)

        def wait_out(ob, sem):
            pltpu.make_async_copy(ob, out_ref.at[pl.ds(0, CD)], sem).wait()

        @pl.when(nk > 0)
        def _():
            fire_idx(0, bufs0, semi0)

        parity = ((bufs0, ob0, semi0, semo0, bufs1, semi1),
                  (bufs1, ob1, semi1, semo1, bufs0, semi0))

        def pair(t, _):
            for p in range(2):
                bufs, ob, semi, semo, nbufs, nsemi = parity[p]
                k = 2 * t + p

                @pl.when(k < nk)
                def _():
                    wait_idx(bufs, semi)

                    @pl.when(k + 1 < nk)
                    def _():
                        fire_idx(k + 1, nbufs, nsemi)

                    @pl.when(k >= 2)
                    def _():
                        wait_out(ob, semo)

                    compute(bufs, ob)
                    fire_out(k, ob, semo)
            return 0

        lax.fori_loop(0, (nk + 1) // 2, pair, 0)

        for p in range(2):
            _, ob, _, semo, _, _ = parity[p]

            @pl.when(((nk >= 1) & ((nk - 1) % 2 == p))
                     | ((nk >= 2) & ((nk - 2) % 2 == p)))
            def _():
                wait_out(ob, semo)

    def compute_atom(bufs, ob):
        a0, a1, a2, a3, a4, a5 = bufs

        @plsc.parallel_loop(0, CA // L)
        def group(g):
            sl = pl.ds(g * L, L)
            atv = a0[sl] * DA
            c1v = ((a1[sl] + 1) * 9 + a2[sl]) * DA
            c2v = ((a3[sl] * 8 + a4[sl]) * 4 + a5[sl]) * DA
            for j in range(L):
                s0 = jnp.full((L,), atv[j], jnp.int32)
                s1 = jnp.full((L,), c1v[j], jnp.int32)
                s2 = jnp.full((L,), c2v[j], jnp.int32)
                base = (g * L + j) * DA
                for k in range(DA // L):
                    ik = iota16 + (k * L)
                    v = (plsc.load_gather(v_atf, [s0 + ik])
                         + plsc.load_gather(f1a, [s1 + ik])
                         + plsc.load_gather(f2a, [s2 + ik]))
                    ob[pl.ds(base + k * L, L)] = v

    run_phase(NCHUNK_A, CA, CA * DA,
              (i_at, i_fc, i_nh, i_ar, i_hy, i_ch),
              (aA0, aA1, aA2, aA3, aA4, aA5),
              (aB0, aB1, aB2, aB3, aB4, aB5),
              atom_out, oaA, oaB, semiA, semiB, semoA, semoB, compute_atom)

    def compute_bond(bufs, ob):
        b0, b1, b2, b3, b4 = bufs

        @plsc.parallel_loop(0, CB // L)
        def group(g):
            sl = pl.ds(g * L, L)
            c1v = (b0[sl] * 8 + b1[sl]) * DB
            c2v = ((b2[sl] * 2 + b3[sl]) * 32 + b4[sl]) * DB
            for j in range(L):
                s1 = jnp.full((L,), c1v[j], jnp.int32)
                s2 = jnp.full((L,), c2v[j], jnp.int32)
                base = (g * L + j) * DB
                for k in range(DB // L):
                    ik = iota16 + (k * L)
                    v = (plsc.load_gather(f1b, [s1 + ik])
                         + plsc.load_gather(f2b, [s2 + ik]))
                    ob[pl.ds(base + k * L, L)] = v

    run_phase(NCHUNK_B, CB, CB * DB,
              (i_bt, i_st, i_cj, i_ir, i_gd),
              (bA0, bA1, bA2, bA3, bA4),
              (bB0, bB1, bB2, bB3, bB4),
              bond_out, obA, obB, semiA, semiB, semoA, semoB, compute_bond)


_sc_call = functools.partial(
    pl.kernel,
    out_type=(jax.ShapeDtypeStruct((NA * DA,), jnp.float32),
              jax.ShapeDtypeStruct((NB * DB,), jnp.float32)),
    mesh=plsc.VectorSubcoreMesh(core_axis_name="c", subcore_axis_name="s",
                                num_cores=NC, num_subcores=NS),
    compiler_params=pltpu.CompilerParams(needs_layout_passes=False),
    scratch_types=[
        pltpu.VMEM((100, DA), jnp.float32),
        pltpu.VMEM((100 * DA,), jnp.float32),
        pltpu.VMEM((8, DA), jnp.float32),
        pltpu.VMEM((9, DA), jnp.float32),
        pltpu.VMEM((2, DA), jnp.float32),
        pltpu.VMEM((8, DA), jnp.float32),
        pltpu.VMEM((4, DA), jnp.float32),
        pltpu.VMEM((8, DB), jnp.float32),
        pltpu.VMEM((8, DB), jnp.float32),
        pltpu.VMEM((2, DB), jnp.float32),
        pltpu.VMEM((2, DB), jnp.float32),
        pltpu.VMEM((32, DB), jnp.float32),
        pltpu.VMEM((72 * DA,), jnp.float32),
        pltpu.VMEM((64 * DA,), jnp.float32),
        pltpu.VMEM((64 * DB,), jnp.float32),
        pltpu.VMEM((128 * DB,), jnp.float32),
    ] + [pltpu.VMEM((CA,), jnp.int32)] * 12
      + [pltpu.VMEM((CB,), jnp.int32)] * 10
      + [
        pltpu.VMEM((CA * DA,), jnp.float32),
        pltpu.VMEM((CA * DA,), jnp.float32),
        pltpu.VMEM((CB * DB,), jnp.float32),
        pltpu.VMEM((CB * DB,), jnp.float32),
        pltpu.SemaphoreType.DMA,
        pltpu.SemaphoreType.DMA,
        pltpu.SemaphoreType.DMA,
        pltpu.SemaphoreType.DMA,
    ],
)(_sc_body)


@jax.jit
def kernel(atom_type, formal_charge, num_H, aromaticity, hybridization,
           chiral, bond_type, stereo, conjugated, in_ring, graph_distance,
           W_atom_type, W_formal_charge, W_num_H, W_aromaticity,
           W_hybridization, W_chiral, W_bond_type, W_stereo, W_conjugated,
           W_in_ring, W_graph_distance):
    ii = [atom_type, formal_charge, num_H, aromaticity, hybridization,
          chiral, bond_type, stereo, conjugated, in_ring, graph_distance]
    ii = [x.astype(jnp.int32) for x in ii]
    atom_flat, bond_flat = _sc_call(
        *ii,
        W_atom_type, W_formal_charge, W_num_H, W_aromaticity,
        W_hybridization, W_chiral,
        W_bond_type, W_stereo, W_conjugated, W_in_ring, W_graph_distance)
    return atom_flat.reshape(NA, DA), bond_flat.reshape(NB, DB)


# speedup vs baseline: 10.7969x; 1.0020x over previous
import functools
import jax
import jax.numpy as jnp
from jax import lax
from jax.experimental import pallas as pl
from jax.experimental.pallas import tpu as pltpu
from jax.experimental.pallas import tpu_sc as plsc
from jax.experimental import layout as jex_layout

NC, NS, L = 2, 16, 16
NW = NC * NS
NA, NB = 50000, 800000
DA, DB = 128, 64
CA, CB = 80, 256
NCHUNK_A = NA // CA
NCHUNK_B = NB // CB


def _build_fused(dst, nrows, d, terms):
    def body(r, _):
        for s in range(d // L):
            sl = pl.ds(s * L, L)
            acc = terms[0][0][terms[0][1](r), sl]
            for ref, fn in terms[1:]:
                acc = acc + ref[fn(r), sl]
            dst[pl.ds(r * d + s * L, L)] = acc
        return 0
    lax.fori_loop(0, nrows, body, 0)


def _sc_body(i_at, i_fc, i_nh, i_ar, i_hy, i_ch,
             i_bt, i_st, i_cj, i_ir, i_gd,
             w_at, w_fc, w_nh, w_ar, w_hy, w_ch,
             w_bt, w_st, w_cj, w_ir, w_gd,
             atom_out, bond_out,
             v_at, v_atf, v_fc, v_nh, v_ar, v_hy, v_ch,
             v_bt, v_st, v_cj, v_ir, v_gd,
             f1a, f2a, f1b, f2b,
             aA0, aA1, aA2, aA3, aA4, aA5,
             aB0, aB1, aB2, aB3, aB4, aB5,
             bA0, bA1, bA2, bA3, bA4,
             bB0, bB1, bB2, bB3, bB4,
             oaA, oaB, obA, obB,
             semiA, semiB, semoA, semoB):
    wid = lax.axis_index("s") * NC + lax.axis_index("c")
    iota16 = lax.broadcasted_iota(jnp.int32, (L,), 0)

    pltpu.sync_copy(w_at, v_at)
    pltpu.sync_copy(w_fc, v_fc)
    pltpu.sync_copy(w_nh, v_nh)
    pltpu.sync_copy(w_ar, v_ar)
    pltpu.sync_copy(w_hy, v_hy)
    pltpu.sync_copy(w_ch, v_ch)
    pltpu.sync_copy(w_bt, v_bt)
    pltpu.sync_copy(w_st, v_st)
    pltpu.sync_copy(w_cj, v_cj)
    pltpu.sync_copy(w_ir, v_ir)
    pltpu.sync_copy(w_gd, v_gd)

    def at_row(r, _):
        for s in range(DA // L):
            v_atf[pl.ds(r * DA + s * L, L)] = v_at[r, pl.ds(s * L, L)]
        return 0
    lax.fori_loop(0, 100, at_row, 0)

    _build_fused(f1a, 72, DA, [(v_fc, lambda r: r // 9),
                               (v_nh, lambda r: r % 9)])
    _build_fused(f2a, 64, DA, [(v_ar, lambda r: r // 32),
                               (v_hy, lambda r: (r // 4) % 8),
                               (v_ch, lambda r: r % 4)])
    _build_fused(f1b, 64, DB, [(v_bt, lambda r: r // 8),
                               (v_st, lambda r: r % 8)])
    _build_fused(f2b, 128, DB, [(v_cj, lambda r: r // 64),
                                (v_ir, lambda r: (r // 32) % 2),
                                (v_gd, lambda r: r % 32)])

    def run_phase(nchunks, C, CD, idx_srcs, bufs0, bufs1, out_ref,
                  ob0, ob1, semi0, semi1, semo0, semo1, compute):
        nk = (nchunks - wid + NW - 1) // NW

        def fire_idx(k, bufs, sem):
            sl = pl.ds((wid + NW * k) * C, C)
            for s, d in zip(idx_srcs, bufs):
                pltpu.async_copy(s.at[sl], d, sem)

        def wait_idx(bufs, sem):
            for s, d in zip(idx_srcs, bufs):
                pltpu.make_async_copy(s.at[pl.ds(0, C)], d, sem).wait()

        def fire_out(k, ob, sem):
            pltpu.async_copy(ob, out_ref.at[pl.ds((wid + NW * k) * CD, CD)],
                             sem)

        def wait_out(ob, sem):
            pltpu.make_async_copy(ob, out_ref.at[pl.ds(0, CD)], sem).wait()

        @pl.when(nk > 0)
        def _():
            fire_idx(0, bufs0, semi0)

        parity = ((bufs0, ob0, semi0, semo0, bufs1, semi1),
                  (bufs1, ob1, semi1, semo1, bufs0, semi0))

        def pair(t, _):
            for p in range(2):
                bufs, ob, semi, semo, nbufs, nsemi = parity[p]
                k = 2 * t + p

                @pl.when(k < nk)
                def _():
                    wait_idx(bufs, semi)

                    @pl.when(k + 1 < nk)
                    def _():
                        fire_idx(k + 1, nbufs, nsemi)

                    @pl.when(k >= 2)
                    def _():
                        wait_out(ob, semo)

                    compute(bufs, ob)
                    fire_out(k, ob, semo)
            return 0

        lax.fori_loop(0, (nk + 1) // 2, pair, 0)

        for p in range(2):
            _, ob, _, semo, _, _ = parity[p]

            @pl.when(((nk >= 1) & ((nk - 1) % 2 == p))
                     | ((nk >= 2) & ((nk - 2) % 2 == p)))
            def _():
                wait_out(ob, semo)

    def compute_atom(bufs, ob):
        a0, a1, a2, a3, a4, a5 = bufs

        @plsc.parallel_loop(0, CA // L)
        def group(g):
            sl = pl.ds(g * L, L)
            atv = a0[sl] * DA
            c1v = ((a1[sl] + 1) * 9 + a2[sl]) * DA
            c2v = ((a3[sl] * 8 + a4[sl]) * 4 + a5[sl]) * DA
            for j in range(L):
                s0 = jnp.full((L,), atv[j], jnp.int32)
                s1 = jnp.full((L,), c1v[j], jnp.int32)
                s2 = jnp.full((L,), c2v[j], jnp.int32)
                base = (g * L + j) * DA
                for k in range(DA // L):
                    ik = iota16 + (k * L)
                    v = (plsc.load_gather(v_atf, [s0 + ik])
                         + plsc.load_gather(f1a, [s1 + ik])
                         + plsc.load_gather(f2a, [s2 + ik]))
                    ob[pl.ds(base + k * L, L)] = v

    run_phase(NCHUNK_A, CA, CA * DA,
              (i_at, i_fc, i_nh, i_ar, i_hy, i_ch),
              (aA0, aA1, aA2, aA3, aA4, aA5),
              (aB0, aB1, aB2, aB3, aB4, aB5),
              atom_out, oaA, oaB, semiA, semiB, semoA, semoB, compute_atom)

    def compute_bond(bufs, ob):
        b0, b1, b2, b3, b4 = bufs

        @plsc.parallel_loop(0, CB // L)
        def group(g):
            sl = pl.ds(g * L, L)
            c1v = (b0[sl] * 8 + b1[sl]) * DB
            c2v = ((b2[sl] * 2 + b3[sl]) * 32 + b4[sl]) * DB
            for j in range(L):
                s1 = jnp.full((L,), c1v[j], jnp.int32)
                s2 = jnp.full((L,), c2v[j], jnp.int32)
                base = (g * L + j) * DB
                for k in range(DB // L):
                    ik = iota16 + (k * L)
                    v = (plsc.load_gather(f1b, [s1 + ik])
                         + plsc.load_gather(f2b, [s2 + ik]))
                    ob[pl.ds(base + k * L, L)] = v

    run_phase(NCHUNK_B, CB, CB * DB,
              (i_bt, i_st, i_cj, i_ir, i_gd),
              (bA0, bA1, bA2, bA3, bA4),
              (bB0, bB1, bB2, bB3, bB4),
              bond_out, obA, obB, semiA, semiB, semoA, semoB, compute_bond)


_sc_call = functools.partial(
    pl.kernel,
    out_type=(jax.ShapeDtypeStruct((NA * DA,), jnp.float32),
              jax.ShapeDtypeStruct((NB * DB,), jnp.float32)),
    mesh=plsc.VectorSubcoreMesh(core_axis_name="c", subcore_axis_name="s",
                                num_cores=NC, num_subcores=NS),
    compiler_params=pltpu.CompilerParams(needs_layout_passes=False),
    scratch_types=[
        pltpu.VMEM((100, DA), jnp.float32),
        pltpu.VMEM((100 * DA,), jnp.float32),
        pltpu.VMEM((8, DA), jnp.float32),
        pltpu.VMEM((9, DA), jnp.float32),
        pltpu.VMEM((2, DA), jnp.float32),
        pltpu.VMEM((8, DA), jnp.float32),
        pltpu.VMEM((4, DA), jnp.float32),
        pltpu.VMEM((8, DB), jnp.float32),
        pltpu.VMEM((8, DB), jnp.float32),
        pltpu.VMEM((2, DB), jnp.float32),
        pltpu.VMEM((2, DB), jnp.float32),
        pltpu.VMEM((32, DB), jnp.float32),
        pltpu.VMEM((72 * DA,), jnp.float32),
        pltpu.VMEM((64 * DA,), jnp.float32),
        pltpu.VMEM((64 * DB,), jnp.float32),
        pltpu.VMEM((128 * DB,), jnp.float32),
    ] + [pltpu.VMEM((CA,), jnp.int32)] * 12
      + [pltpu.VMEM((CB,), jnp.int32)] * 10
      + [
        pltpu.VMEM((CA * DA,), jnp.float32),
        pltpu.VMEM((CA * DA,), jnp.float32),
        pltpu.VMEM((CB * DB,), jnp.float32),
        pltpu.VMEM((CB * DB,), jnp.float32),
        pltpu.SemaphoreType.DMA,
        pltpu.SemaphoreType.DMA,
        pltpu.SemaphoreType.DMA,
        pltpu.SemaphoreType.DMA,
    ],
)(_sc_body)


def _impl(atom_type, formal_charge, num_H, aromaticity, hybridization,
          chiral, bond_type, stereo, conjugated, in_ring, graph_distance,
          W_atom_type, W_formal_charge, W_num_H, W_aromaticity,
          W_hybridization, W_chiral, W_bond_type, W_stereo, W_conjugated,
          W_in_ring, W_graph_distance):
    ii = [atom_type, formal_charge, num_H, aromaticity, hybridization,
          chiral, bond_type, stereo, conjugated, in_ring, graph_distance]
    ii = [x.astype(jnp.int32) for x in ii]
    atom_flat, bond_flat = _sc_call(
        *ii,
        W_atom_type, W_formal_charge, W_num_H, W_aromaticity,
        W_hybridization, W_chiral,
        W_bond_type, W_stereo, W_conjugated, W_in_ring, W_graph_distance)
    return atom_flat.reshape(NA, DA), bond_flat.reshape(NB, DB)


_jitted = []


def kernel(*args):
    if not _jitted:
        sh = None
        for a in args:
            if isinstance(getattr(a, "sharding", None), jax.sharding.Sharding):
                sh = a.sharding
                break
        if sh is None:
            sh = jax.sharding.SingleDeviceSharding(jax.devices()[0])
        fmt = jex_layout.Format(jex_layout.Layout(major_to_minor=(0, 1)), sh)
        _jitted.append(jax.jit(_impl, out_shardings=(fmt, fmt)))
    return _jitted[0](*args)


# speedup vs baseline: 16.8172x; 1.5576x over previous
import functools
import jax
import jax.numpy as jnp
from jax import lax
from jax.experimental import pallas as pl
from jax.experimental.pallas import tpu as pltpu
from jax.experimental.pallas import tpu_sc as plsc
from jax.experimental import layout as jex_layout

NC, NS, L = 2, 16, 16
NW = NC * NS
NA, NB = 50000, 800000
DA, DB = 128, 64
CA, CB = 80, 128
PB = DB + 1
NCHUNK_A = NA // CA
NCHUNK_B = NB // CB


def _build_fused(dst, nrows, d, terms):
    def body(r, _):
        for s in range(d // L):
            sl = pl.ds(s * L, L)
            acc = terms[0][0][terms[0][1](r), sl]
            for ref, fn in terms[1:]:
                acc = acc + ref[fn(r), sl]
            dst[pl.ds(r * d + s * L, L)] = acc
        return 0
    lax.fori_loop(0, nrows, body, 0)


def _sc_body(i_at, i_fc, i_nh, i_ar, i_hy, i_ch,
             i_bt, i_st, i_cj, i_ir, i_gd,
             w_at, w_fc, w_nh, w_ar, w_hy, w_ch,
             w_bt, w_st, w_cj, w_ir, w_gd,
             atom_out, bond_out,
             v_at, v_atf, v_fc, v_nh, v_ar, v_hy, v_ch,
             v_bt, v_st, v_cj, v_ir, v_gd,
             f1a, f2a, f1b, f2b,
             aA0, aA1, aA2, aA3, aA4, aA5,
             aB0, aB1, aB2, aB3, aB4, aB5,
             bA0, bA1, bA2, bA3, bA4,
             bB0, bB1, bB2, bB3, bB4,
             oaA, oaB, obA, obB, v_pad,
             semiA, semiB, semoA, semoB):
    wid = lax.axis_index("s") * NC + lax.axis_index("c")
    iota16 = lax.broadcasted_iota(jnp.int32, (L,), 0)

    pltpu.sync_copy(w_at, v_at)
    pltpu.sync_copy(w_fc, v_fc)
    pltpu.sync_copy(w_nh, v_nh)
    pltpu.sync_copy(w_ar, v_ar)
    pltpu.sync_copy(w_hy, v_hy)
    pltpu.sync_copy(w_ch, v_ch)
    pltpu.sync_copy(w_bt, v_bt)
    pltpu.sync_copy(w_st, v_st)
    pltpu.sync_copy(w_cj, v_cj)
    pltpu.sync_copy(w_ir, v_ir)
    pltpu.sync_copy(w_gd, v_gd)

    def at_row(r, _):
        for s in range(DA // L):
            v_atf[pl.ds(r * DA + s * L, L)] = v_at[r, pl.ds(s * L, L)]
        return 0
    lax.fori_loop(0, 100, at_row, 0)

    _build_fused(f1a, 72, DA, [(v_fc, lambda r: r // 9),
                               (v_nh, lambda r: r % 9)])
    _build_fused(f2a, 64, DA, [(v_ar, lambda r: r // 32),
                               (v_hy, lambda r: (r // 4) % 8),
                               (v_ch, lambda r: r % 4)])
    _build_fused(f1b, 64, DB, [(v_bt, lambda r: r // 8),
                               (v_st, lambda r: r % 8)])
    _build_fused(f2b, 128, DB, [(v_cj, lambda r: r // 64),
                                (v_ir, lambda r: (r // 32) % 2),
                                (v_gd, lambda r: r % 32)])

    def run_phase(nchunks, C, out_slice, idx_srcs, bufs0, bufs1, out_ref,
                  ob0, ob1, semi0, semi1, semo0, semo1, compute):
        nk = (nchunks - wid + NW - 1) // NW

        def fire_idx(k, bufs, sem):
            sl = pl.ds((wid + NW * k) * C, C)
            for s, d in zip(idx_srcs, bufs):
                pltpu.async_copy(s.at[sl], d, sem)

        def wait_idx(bufs, sem):
            for s, d in zip(idx_srcs, bufs):
                pltpu.make_async_copy(s.at[pl.ds(0, C)], d, sem).wait()

        def fire_out(k, ob, sem):
            pltpu.async_copy(ob, out_slice(out_ref, wid + NW * k), sem)

        def wait_out(ob, sem):
            pltpu.make_async_copy(ob, out_slice(out_ref, 0), sem).wait()

        @pl.when(nk > 0)
        def _():
            fire_idx(0, bufs0, semi0)

        parity = ((bufs0, ob0, semi0, semo0, bufs1, semi1),
                  (bufs1, ob1, semi1, semo1, bufs0, semi0))

        def pair(t, _):
            for p in range(2):
                bufs, ob, semi, semo, nbufs, nsemi = parity[p]
                k = 2 * t + p

                @pl.when(k < nk)
                def _():
                    wait_idx(bufs, semi)

                    @pl.when(k + 1 < nk)
                    def _():
                        fire_idx(k + 1, nbufs, nsemi)

                    @pl.when(k >= 2)
                    def _():
                        wait_out(ob, semo)

                    compute(bufs, ob)
                    fire_out(k, ob, semo)
            return 0

        lax.fori_loop(0, (nk + 1) // 2, pair, 0)

        for p in range(2):
            _, ob, _, semo, _, _ = parity[p]

            @pl.when(((nk >= 1) & ((nk - 1) % 2 == p))
                     | ((nk >= 2) & ((nk - 2) % 2 == p)))
            def _():
                wait_out(ob, semo)

    def compute_atom(bufs, ob):
        a0, a1, a2, a3, a4, a5 = bufs

        @plsc.parallel_loop(0, CA // L)
        def group(g):
            sl = pl.ds(g * L, L)
            atv = a0[sl] * DA
            c1v = ((a1[sl] + 1) * 9 + a2[sl]) * DA
            c2v = ((a3[sl] * 8 + a4[sl]) * 4 + a5[sl]) * DA
            for j in range(L):
                s0 = jnp.full((L,), atv[j], jnp.int32)
                s1 = jnp.full((L,), c1v[j], jnp.int32)
                s2 = jnp.full((L,), c2v[j], jnp.int32)
                base = (g * L + j) * DA
                for k in range(DA // L):
                    ik = iota16 + (k * L)
                    v = (plsc.load_gather(v_atf, [s0 + ik])
                         + plsc.load_gather(f1a, [s1 + ik])
                         + plsc.load_gather(f2a, [s2 + ik]))
                    ob[pl.ds(base + k * L, L)] = v

    run_phase(NCHUNK_A, CA,
              lambda ref, c: ref.at[pl.ds(c * (CA * DA), CA * DA)],
              (i_at, i_fc, i_nh, i_ar, i_hy, i_ch),
              (aA0, aA1, aA2, aA3, aA4, aA5),
              (aB0, aB1, aB2, aB3, aB4, aB5),
              atom_out, oaA, oaB, semiA, semiB, semoA, semoB, compute_atom)

    iota_pb = iota16 * PB

    def compute_bond(bufs, ob):
        b0, b1, b2, b3, b4 = bufs

        @plsc.parallel_loop(0, CB // L)
        def group(g):
            sl = pl.ds(g * L, L)
            c1v = (b0[sl] * 8 + b1[sl]) * DB
            c2v = ((b2[sl] * 2 + b3[sl]) * 32 + b4[sl]) * DB
            for j in range(L):
                s1 = jnp.full((L,), c1v[j], jnp.int32)
                s2 = jnp.full((L,), c2v[j], jnp.int32)
                base = (g * L + j) * PB
                for k in range(DB // L):
                    ik = iota16 + (k * L)
                    v = (plsc.load_gather(f1b, [s1 + ik])
                         + plsc.load_gather(f2b, [s2 + ik]))
                    v_pad[pl.ds(base + k * L, L)] = v

        @plsc.parallel_loop(0, DB)
        def drow(d):
            for t in range(CB // L):
                idx = iota_pb + (t * L * PB + d)
                ob[d, pl.ds(t * L, L)] = plsc.load_gather(v_pad, [idx])

    run_phase(NCHUNK_B, CB,
              lambda ref, c: ref.at[:, pl.ds(c * CB, CB)],
              (i_bt, i_st, i_cj, i_ir, i_gd),
              (bA0, bA1, bA2, bA3, bA4),
              (bB0, bB1, bB2, bB3, bB4),
              bond_out, obA, obB, semiA, semiB, semoA, semoB, compute_bond)


_sc_call = functools.partial(
    pl.kernel,
    out_type=(jax.ShapeDtypeStruct((NA * DA,), jnp.float32),
              jax.ShapeDtypeStruct((DB, NB), jnp.float32)),
    mesh=plsc.VectorSubcoreMesh(core_axis_name="c", subcore_axis_name="s",
                                num_cores=NC, num_subcores=NS),
    compiler_params=pltpu.CompilerParams(needs_layout_passes=False),
    scratch_types=[
        pltpu.VMEM((100, DA), jnp.float32),
        pltpu.VMEM((100 * DA,), jnp.float32),
        pltpu.VMEM((8, DA), jnp.float32),
        pltpu.VMEM((9, DA), jnp.float32),
        pltpu.VMEM((2, DA), jnp.float32),
        pltpu.VMEM((8, DA), jnp.float32),
        pltpu.VMEM((4, DA), jnp.float32),
        pltpu.VMEM((8, DB), jnp.float32),
        pltpu.VMEM((8, DB), jnp.float32),
        pltpu.VMEM((2, DB), jnp.float32),
        pltpu.VMEM((2, DB), jnp.float32),
        pltpu.VMEM((32, DB), jnp.float32),
        pltpu.VMEM((72 * DA,), jnp.float32),
        pltpu.VMEM((64 * DA,), jnp.float32),
        pltpu.VMEM((64 * DB,), jnp.float32),
        pltpu.VMEM((128 * DB,), jnp.float32),
    ] + [pltpu.VMEM((CA,), jnp.int32)] * 12
      + [pltpu.VMEM((CB,), jnp.int32)] * 10
      + [
        pltpu.VMEM((CA * DA,), jnp.float32),
        pltpu.VMEM((CA * DA,), jnp.float32),
        pltpu.VMEM((DB, CB), jnp.float32),
        pltpu.VMEM((DB, CB), jnp.float32),
        pltpu.VMEM((CB * PB,), jnp.float32),
        pltpu.SemaphoreType.DMA,
        pltpu.SemaphoreType.DMA,
        pltpu.SemaphoreType.DMA,
        pltpu.SemaphoreType.DMA,
    ],
)(_sc_body)


def _impl(atom_type, formal_charge, num_H, aromaticity, hybridization,
          chiral, bond_type, stereo, conjugated, in_ring, graph_distance,
          W_atom_type, W_formal_charge, W_num_H, W_aromaticity,
          W_hybridization, W_chiral, W_bond_type, W_stereo, W_conjugated,
          W_in_ring, W_graph_distance):
    ii = [atom_type, formal_charge, num_H, aromaticity, hybridization,
          chiral, bond_type, stereo, conjugated, in_ring, graph_distance]
    ii = [x.astype(jnp.int32) for x in ii]
    atom_flat, bond_t = _sc_call(
        *ii,
        W_atom_type, W_formal_charge, W_num_H, W_aromaticity,
        W_hybridization, W_chiral,
        W_bond_type, W_stereo, W_conjugated, W_in_ring, W_graph_distance)
    return atom_flat.reshape(NA, DA), bond_t.T


kernel = jax.jit(_impl)


# speedup vs baseline: 26.6758x; 1.5862x over previous
import functools
import jax
import jax.numpy as jnp
from jax import lax
from jax.experimental import pallas as pl
from jax.experimental.pallas import tpu as pltpu
from jax.experimental.pallas import tpu_sc as plsc
from jax.experimental import layout as jex_layout

NC, NS, L = 2, 16, 16
NW = NC * NS
NA, NB = 50000, 800000
DA, DB = 128, 64
CA, CB = 80, 128
WA, WB = DA // 2, DB // 2
PB = WB + 1
NCHUNK_A = NA // CA
NCHUNK_B = NB // CB


def _build_packed(dst, nrows, d, terms):
    w = d // 2
    def body(r, _):
        for s in range(w // L):
            lo_sl = pl.ds(s * L, L)
            hi_sl = pl.ds(w + s * L, L)
            lo = terms[0][0][terms[0][1](r), lo_sl]
            hi = terms[0][0][terms[0][1](r), hi_sl]
            for ref, fn in terms[1:]:
                lo = lo + ref[fn(r), lo_sl]
                hi = hi + ref[fn(r), hi_sl]
            packed = plsc.bitcast(
                plsc.pack(lo, hi, format=plsc.PackFormat.INTERLEAVED),
                jnp.int32)
            dst[pl.ds(r * w + s * L, L)] = packed
        return 0
    lax.fori_loop(0, nrows, body, 0)


def _sc_body(i_at, i_fc, i_nh, i_ar, i_hy, i_ch,
             i_bt, i_st, i_cj, i_ir, i_gd,
             w_at, w_fc, w_nh, w_ar, w_hy, w_ch,
             w_bt, w_st, w_cj, w_ir, w_gd,
             atom_out, bond_out,
             v_at, v_atf, v_fc, v_nh, v_ar, v_hy, v_ch,
             v_bt, v_st, v_cj, v_ir, v_gd,
             f1a, f2a, f1b, f2b,
             aA0, aA1, aA2, aA3, aA4, aA5,
             aB0, aB1, aB2, aB3, aB4, aB5,
             bA0, bA1, bA2, bA3, bA4,
             bB0, bB1, bB2, bB3, bB4,
             oaA, oaB, obA, obB, v_pad,
             semiA, semiB, semoA, semoB):
    wid = lax.axis_index("s") * NC + lax.axis_index("c")
    iota16 = lax.broadcasted_iota(jnp.int32, (L,), 0)

    pltpu.sync_copy(w_at, v_at)
    pltpu.sync_copy(w_fc, v_fc)
    pltpu.sync_copy(w_nh, v_nh)
    pltpu.sync_copy(w_ar, v_ar)
    pltpu.sync_copy(w_hy, v_hy)
    pltpu.sync_copy(w_ch, v_ch)
    pltpu.sync_copy(w_bt, v_bt)
    pltpu.sync_copy(w_st, v_st)
    pltpu.sync_copy(w_cj, v_cj)
    pltpu.sync_copy(w_ir, v_ir)
    pltpu.sync_copy(w_gd, v_gd)

    _build_packed(v_atf, 100, DA, [(v_at, lambda r: r)])
    _build_packed(f1a, 72, DA, [(v_fc, lambda r: r // 9),
                                (v_nh, lambda r: r % 9)])
    _build_packed(f2a, 64, DA, [(v_ar, lambda r: r // 32),
                                (v_hy, lambda r: (r // 4) % 8),
                                (v_ch, lambda r: r % 4)])
    _build_packed(f1b, 64, DB, [(v_bt, lambda r: r // 8),
                                (v_st, lambda r: r % 8)])
    _build_packed(f2b, 128, DB, [(v_cj, lambda r: r // 64),
                                 (v_ir, lambda r: (r // 32) % 2),
                                 (v_gd, lambda r: r % 32)])

    def _unpack_lo(wi):
        return plsc.bitcast(jnp.left_shift(wi, 16), jnp.float32)

    def _unpack_hi(wi):
        return plsc.bitcast(jnp.bitwise_and(wi, jnp.int32(-65536)),
                            jnp.float32)

    def run_phase(nchunks, C, out_slice, idx_srcs, bufs0, bufs1, out_ref,
                  ob0, ob1, semi0, semi1, semo0, semo1, compute):
        nk = (nchunks - wid + NW - 1) // NW

        def fire_idx(k, bufs, sem):
            sl = pl.ds((wid + NW * k) * C, C)
            for s, d in zip(idx_srcs, bufs):
                pltpu.async_copy(s.at[sl], d, sem)

        def wait_idx(bufs, sem):
            for s, d in zip(idx_srcs, bufs):
                pltpu.make_async_copy(s.at[pl.ds(0, C)], d, sem).wait()

        def fire_out(k, ob, sem):
            pltpu.async_copy(ob, out_slice(out_ref, wid + NW * k), sem)

        def wait_out(ob, sem):
            pltpu.make_async_copy(ob, out_slice(out_ref, 0), sem).wait()

        @pl.when(nk > 0)
        def _():
            fire_idx(0, bufs0, semi0)

        parity = ((bufs0, ob0, semi0, semo0, bufs1, semi1),
                  (bufs1, ob1, semi1, semo1, bufs0, semi0))

        def pair(t, _):
            for p in range(2):
                bufs, ob, semi, semo, nbufs, nsemi = parity[p]
                k = 2 * t + p

                @pl.when(k < nk)
                def _():
                    wait_idx(bufs, semi)

                    @pl.when(k + 1 < nk)
                    def _():
                        fire_idx(k + 1, nbufs, nsemi)

                    @pl.when(k >= 2)
                    def _():
                        wait_out(ob, semo)

                    compute(bufs, ob)
                    fire_out(k, ob, semo)
            return 0

        lax.fori_loop(0, (nk + 1) // 2, pair, 0)

        for p in range(2):
            _, ob, _, semo, _, _ = parity[p]

            @pl.when(((nk >= 1) & ((nk - 1) % 2 == p))
                     | ((nk >= 2) & ((nk - 2) % 2 == p)))
            def _():
                wait_out(ob, semo)

    def compute_atom(bufs, ob):
        a0, a1, a2, a3, a4, a5 = bufs

        @plsc.parallel_loop(0, CA // L)
        def group(g):
            sl = pl.ds(g * L, L)
            atv = a0[sl] * WA
            c1v = ((a1[sl] + 1) * 9 + a2[sl]) * WA
            c2v = ((a3[sl] * 8 + a4[sl]) * 4 + a5[sl]) * WA
            for j in range(L):
                s0 = jnp.full((L,), atv[j], jnp.int32)
                s1 = jnp.full((L,), c1v[j], jnp.int32)
                s2 = jnp.full((L,), c2v[j], jnp.int32)
                base = (g * L + j) * DA
                for k in range(WA // L):
                    ik = iota16 + (k * L)
                    w = (plsc.bitcast(plsc.load_gather(v_atf, [s0 + ik]),
                                      jnp.bfloat16)
                         + plsc.bitcast(plsc.load_gather(f1a, [s1 + ik]),
                                        jnp.bfloat16)
                         + plsc.bitcast(plsc.load_gather(f2a, [s2 + ik]),
                                        jnp.bfloat16))
                    wi = plsc.bitcast(w, jnp.int32)
                    ob[pl.ds(base + k * L, L)] = _unpack_lo(wi)
                    ob[pl.ds(base + WA + k * L, L)] = _unpack_hi(wi)

    run_phase(NCHUNK_A, CA,
              lambda ref, c: ref.at[pl.ds(c * (CA * DA), CA * DA)],
              (i_at, i_fc, i_nh, i_ar, i_hy, i_ch),
              (aA0, aA1, aA2, aA3, aA4, aA5),
              (aB0, aB1, aB2, aB3, aB4, aB5),
              atom_out, oaA, oaB, semiA, semiB, semoA, semoB, compute_atom)

    iota_pb = iota16 * PB

    def compute_bond(bufs, ob):
        b0, b1, b2, b3, b4 = bufs

        @plsc.parallel_loop(0, CB // L)
        def group(g):
            sl = pl.ds(g * L, L)
            c1v = (b0[sl] * 8 + b1[sl]) * WB
            c2v = ((b2[sl] * 2 + b3[sl]) * 32 + b4[sl]) * WB
            for j in range(L):
                s1 = jnp.full((L,), c1v[j], jnp.int32)
                s2 = jnp.full((L,), c2v[j], jnp.int32)
                base = (g * L + j) * PB
                for k in range(WB // L):
                    ik = iota16 + (k * L)
                    w = (plsc.bitcast(plsc.load_gather(f1b, [s1 + ik]),
                                      jnp.bfloat16)
                         + plsc.bitcast(plsc.load_gather(f2b, [s2 + ik]),
                                        jnp.bfloat16))
                    v_pad[pl.ds(base + k * L, L)] = plsc.bitcast(w, jnp.int32)

        @plsc.parallel_loop(0, WB)
        def drow(d):
            for t in range(CB // L):
                idx = iota_pb + (t * L * PB + d)
                wi = plsc.load_gather(v_pad, [idx])
                ob[d, pl.ds(t * L, L)] = _unpack_lo(wi)
                ob[d + WB, pl.ds(t * L, L)] = _unpack_hi(wi)

    run_phase(NCHUNK_B, CB,
              lambda ref, c: ref.at[:, pl.ds(c * CB, CB)],
              (i_bt, i_st, i_cj, i_ir, i_gd),
              (bA0, bA1, bA2, bA3, bA4),
              (bB0, bB1, bB2, bB3, bB4),
              bond_out, obA, obB, semiA, semiB, semoA, semoB, compute_bond)


_sc_call = functools.partial(
    pl.kernel,
    out_type=(jax.ShapeDtypeStruct((NA * DA,), jnp.float32),
              jax.ShapeDtypeStruct((DB, NB), jnp.float32)),
    mesh=plsc.VectorSubcoreMesh(core_axis_name="c", subcore_axis_name="s",
                                num_cores=NC, num_subcores=NS),
    compiler_params=pltpu.CompilerParams(needs_layout_passes=False),
    scratch_types=[
        pltpu.VMEM((100, DA), jnp.float32),
        pltpu.VMEM((100 * WA,), jnp.int32),
        pltpu.VMEM((8, DA), jnp.float32),
        pltpu.VMEM((9, DA), jnp.float32),
        pltpu.VMEM((2, DA), jnp.float32),
        pltpu.VMEM((8, DA), jnp.float32),
        pltpu.VMEM((4, DA), jnp.float32),
        pltpu.VMEM((8, DB), jnp.float32),
        pltpu.VMEM((8, DB), jnp.float32),
        pltpu.VMEM((2, DB), jnp.float32),
        pltpu.VMEM((2, DB), jnp.float32),
        pltpu.VMEM((32, DB), jnp.float32),
        pltpu.VMEM((72 * WA,), jnp.int32),
        pltpu.VMEM((64 * WA,), jnp.int32),
        pltpu.VMEM((64 * WB,), jnp.int32),
        pltpu.VMEM((128 * WB,), jnp.int32),
    ] + [pltpu.VMEM((CA,), jnp.int32)] * 12
      + [pltpu.VMEM((CB,), jnp.int32)] * 10
      + [
        pltpu.VMEM((CA * DA,), jnp.float32),
        pltpu.VMEM((CA * DA,), jnp.float32),
        pltpu.VMEM((DB, CB), jnp.float32),
        pltpu.VMEM((DB, CB), jnp.float32),
        pltpu.VMEM((CB * PB,), jnp.int32),
        pltpu.SemaphoreType.DMA,
        pltpu.SemaphoreType.DMA,
        pltpu.SemaphoreType.DMA,
        pltpu.SemaphoreType.DMA,
    ],
)(_sc_body)


def _impl(atom_type, formal_charge, num_H, aromaticity, hybridization,
          chiral, bond_type, stereo, conjugated, in_ring, graph_distance,
          W_atom_type, W_formal_charge, W_num_H, W_aromaticity,
          W_hybridization, W_chiral, W_bond_type, W_stereo, W_conjugated,
          W_in_ring, W_graph_distance):
    ii = [atom_type, formal_charge, num_H, aromaticity, hybridization,
          chiral, bond_type, stereo, conjugated, in_ring, graph_distance]
    ii = [x.astype(jnp.int32) for x in ii]
    atom_flat, bond_t = _sc_call(
        *ii,
        W_atom_type, W_formal_charge, W_num_H, W_aromaticity,
        W_hybridization, W_chiral,
        W_bond_type, W_stereo, W_conjugated, W_in_ring, W_graph_distance)
    return atom_flat.reshape(NA, DA), bond_t.T


kernel = jax.jit(_impl)
